# baseline (speedup 1.0000x reference)
import jax
import jax.numpy as jnp
from jax.experimental import pallas as pl
from jax.experimental.pallas import tpu as pltpu

_EPS = 1e-5


def _fwd(x_ref, adj_ref, w_in_ref, b_in_ref, conv_w_ref, conv_b_ref,
         g_ref, be_ref, mu_ref, var_ref, w_out_ref, b_out_ref,
         node_ref, graph_ref, abf_ref, dinv_ref):
    f32 = jnp.float32
    bf16 = jnp.bfloat16
    i = pl.program_id(0)
    cur = jax.lax.rem(i, 2)
    prev = 1 - cur
    cdim = (((0,), (0,)), ((), ()))

    a_f = adj_ref[0]
    n = a_f.shape[0]
    abf_ref[cur] = a_f.astype(bf16)
    deg = jnp.sum(a_f, axis=0, keepdims=True) + 1.0
    dinv_ref[cur] = jnp.transpose(jax.lax.rsqrt(deg), (1, 0))

    a = abf_ref[prev]
    dinv = dinv_ref[prev]
    s = jax.lax.dot_general(a, dinv.astype(bf16), cdim,
                            preferred_element_type=f32) + dinv
    u = dinv * s

    h0 = x_ref[0] @ w_in_ref[...] + b_in_ref[...]
    g0 = h0 @ conv_w_ref[0]
    istd0 = jax.lax.rsqrt(var_ref[0:1] + _EPS)
    p = g0 * istd0 * g_ref[0:1]
    q = (conv_b_ref[0:1] - mu_ref[0:1]) * istd0 * g_ref[0:1] + be_ref[0:1]
    h1 = h0 + jnp.maximum(u * p + q, 0.0)

    v = dinv * (h1 @ conv_w_ref[1])
    agg = jax.lax.dot_general(a, v.astype(bf16), cdim,
                              preferred_element_type=f32) + v
    pre = dinv * agg + conv_b_ref[1:2]
    istd1 = jax.lax.rsqrt(var_ref[1:2] + _EPS)
    bn1 = (pre - mu_ref[1:2]) * istd1 * g_ref[1:2] + be_ref[1:2]
    h2 = h1 + jnp.maximum(bn1, 0.0)

    out = h2 @ w_out_ref[...] + b_out_ref[...]
    node_ref[0] = out
    graph_ref[0] = jnp.sum(out, axis=0, keepdims=True) * (1.0 / n)


def kernel(x, adj, W_in, b_in, conv_W, conv_b, bn_gamma, bn_beta, bn_mean,
           bn_var, W_out, b_out):
    B, T, Cin = x.shape
    N = adj.shape[-1]
    H = W_in.shape[1]
    Cout = W_out.shape[1]
    L = conv_W.shape[0]
    BT = B * T

    rd = lambda i: (jnp.minimum(i, BT - 1), 0, 0)
    wr = lambda i: (jnp.maximum(i - 1, 0), 0, 0)
    rep2 = lambda i: (0, 0)
    rep3 = lambda i: (0, 0, 0)
    node, graph = pl.pallas_call(
        _fwd,
        grid=(BT + 1,),
        in_specs=[
            pl.BlockSpec((1, 1, Cin), wr),
            pl.BlockSpec((1, N, N), rd),
            pl.BlockSpec((Cin, H), rep2),
            pl.BlockSpec((1, H), rep2),
            pl.BlockSpec((L, H, H), rep3),
            pl.BlockSpec((L, H), rep2),
            pl.BlockSpec((L, H), rep2),
            pl.BlockSpec((L, H), rep2),
            pl.BlockSpec((L, H), rep2),
            pl.BlockSpec((L, H), rep2),
            pl.BlockSpec((H, Cout), rep2),
            pl.BlockSpec((1, Cout), rep2),
        ],
        out_specs=[
            pl.BlockSpec((1, N, Cout), wr),
            pl.BlockSpec((1, 1, Cout), wr),
        ],
        out_shape=[
            jax.ShapeDtypeStruct((BT, N, Cout), jnp.float32),
            jax.ShapeDtypeStruct((BT, 1, Cout), jnp.float32),
        ],
        scratch_shapes=[
            pltpu.VMEM((2, N, N), jnp.bfloat16),
            pltpu.VMEM((2, N, 1), jnp.float32),
        ],
    )(x.reshape(BT, 1, Cin), adj.reshape(BT, N, N), W_in, b_in.reshape(1, H),
      conv_W, conv_b, bn_gamma, bn_beta, bn_mean, bn_var, W_out,
      b_out.reshape(1, Cout))
    return node.reshape(B, T, N, Cout), graph.reshape(B, T, Cout)


# speedup vs baseline: 1.1954x; 1.1954x over previous
import jax
import jax.numpy as jnp
from jax.experimental import pallas as pl

_EPS = 1e-5


def _one_graph(a_f, x_row, w_in, b_in, conv_w_ref, conv_b_ref,
               g_ref, be_ref, mu_ref, var_ref, w_out, b_out):
    f32 = jnp.float32
    bf16 = jnp.bfloat16
    cdim = (((0,), (0,)), ((), ()))
    a = a_f.astype(bf16)

    deg = jnp.sum(a_f, axis=0, keepdims=True) + 1.0
    dinv = jnp.transpose(jax.lax.rsqrt(deg), (1, 0))
    s = jax.lax.dot_general(a, dinv.astype(bf16), cdim,
                            preferred_element_type=f32) + dinv
    u = dinv * s

    h0 = x_row @ w_in + b_in
    g0 = h0 @ conv_w_ref[0]
    istd0 = jax.lax.rsqrt(var_ref[0:1] + _EPS)
    p = g0 * istd0 * g_ref[0:1]
    q = (conv_b_ref[0:1] - mu_ref[0:1]) * istd0 * g_ref[0:1] + be_ref[0:1]
    h1 = h0 + jnp.maximum(u * p + q, 0.0)

    v = dinv * (h1 @ conv_w_ref[1])
    agg = jax.lax.dot_general(a, v.astype(bf16), cdim,
                              preferred_element_type=f32) + v
    pre = dinv * agg + conv_b_ref[1:2]
    istd1 = jax.lax.rsqrt(var_ref[1:2] + _EPS)
    bn1 = (pre - mu_ref[1:2]) * istd1 * g_ref[1:2] + be_ref[1:2]
    h2 = h1 + jnp.maximum(bn1, 0.0)

    out = h2 @ w_out + b_out
    return out


def _fwd(x_ref, adj_ref, w_in_ref, b_in_ref, conv_w_ref, conv_b_ref,
         g_ref, be_ref, mu_ref, var_ref, w_out_ref, b_out_ref,
         node_ref, graph_ref):
    n = adj_ref.shape[-1]
    for k in range(adj_ref.shape[0]):
        out = _one_graph(adj_ref[k], x_ref[k], w_in_ref[...], b_in_ref[...],
                         conv_w_ref, conv_b_ref, g_ref, be_ref, mu_ref,
                         var_ref, w_out_ref[...], b_out_ref[...])
        node_ref[k] = out
        graph_ref[k] = jnp.sum(out, axis=0, keepdims=True) * (1.0 / n)


_G = 2


def kernel(x, adj, W_in, b_in, conv_W, conv_b, bn_gamma, bn_beta, bn_mean,
           bn_var, W_out, b_out):
    B, T, Cin = x.shape
    N = adj.shape[-1]
    H = W_in.shape[1]
    Cout = W_out.shape[1]
    L = conv_W.shape[0]
    BT = B * T

    g3 = lambda i: (i, 0, 0)
    rep2 = lambda i: (0, 0)
    rep3 = lambda i: (0, 0, 0)
    node, graph = pl.pallas_call(
        _fwd,
        grid=(BT // _G,),
        in_specs=[
            pl.BlockSpec((_G, 1, Cin), g3),
            pl.BlockSpec((_G, N, N), g3),
            pl.BlockSpec((Cin, H), rep2),
            pl.BlockSpec((1, H), rep2),
            pl.BlockSpec((L, H, H), rep3),
            pl.BlockSpec((L, H), rep2),
            pl.BlockSpec((L, H), rep2),
            pl.BlockSpec((L, H), rep2),
            pl.BlockSpec((L, H), rep2),
            pl.BlockSpec((L, H), rep2),
            pl.BlockSpec((H, Cout), rep2),
            pl.BlockSpec((1, Cout), rep2),
        ],
        out_specs=[
            pl.BlockSpec((_G, N, Cout), g3),
            pl.BlockSpec((_G, 1, Cout), g3),
        ],
        out_shape=[
            jax.ShapeDtypeStruct((BT, N, Cout), jnp.float32),
            jax.ShapeDtypeStruct((BT, 1, Cout), jnp.float32),
        ],
    )(x.reshape(BT, 1, Cin), adj.reshape(BT, N, N), W_in, b_in.reshape(1, H),
      conv_W, conv_b, bn_gamma, bn_beta, bn_mean, bn_var, W_out,
      b_out.reshape(1, Cout))
    return node.reshape(B, T, N, Cout), graph.reshape(B, T, Cout)


# speedup vs baseline: 1.2054x; 1.0083x over previous
import jax
import jax.numpy as jnp
from jax.experimental import pallas as pl

_EPS = 1e-5


def _one_graph(a_f, x_row, w_in, b_in, conv_w_ref, conv_b_ref,
               g_ref, be_ref, mu_ref, var_ref, w_out, b_out):
    f32 = jnp.float32
    bf16 = jnp.bfloat16
    cdim = (((0,), (0,)), ((), ()))
    a = a_f.astype(bf16)

    deg = jnp.sum(a_f, axis=0, keepdims=True) + 1.0
    dinv = jnp.transpose(jax.lax.rsqrt(deg), (1, 0))
    s = jax.lax.dot_general(a, dinv.astype(bf16), cdim,
                            preferred_element_type=f32) + dinv
    u = dinv * s

    h0 = x_row @ w_in + b_in
    g0 = h0 @ conv_w_ref[0]
    istd0 = jax.lax.rsqrt(var_ref[0:1] + _EPS)
    p = g0 * istd0 * g_ref[0:1]
    q = (conv_b_ref[0:1] - mu_ref[0:1]) * istd0 * g_ref[0:1] + be_ref[0:1]
    h1 = h0 + jnp.maximum(u * p + q, 0.0)

    v = dinv * (h1 @ conv_w_ref[1])
    agg = jax.lax.dot_general(a, v.astype(bf16), cdim,
                              preferred_element_type=f32) + v
    pre = dinv * agg + conv_b_ref[1:2]
    istd1 = jax.lax.rsqrt(var_ref[1:2] + _EPS)
    bn1 = (pre - mu_ref[1:2]) * istd1 * g_ref[1:2] + be_ref[1:2]
    h2 = h1 + jnp.maximum(bn1, 0.0)

    out = h2 @ w_out + b_out
    return out


def _fwd(x_ref, adj_ref, w_in_ref, b_in_ref, conv_w_ref, conv_b_ref,
         g_ref, be_ref, mu_ref, var_ref, w_out_ref, b_out_ref,
         node_ref, graph_ref):
    n = adj_ref.shape[-1]
    for k in range(adj_ref.shape[0]):
        out = _one_graph(adj_ref[k], x_ref[k], w_in_ref[...], b_in_ref[...],
                         conv_w_ref, conv_b_ref, g_ref, be_ref, mu_ref,
                         var_ref, w_out_ref[...], b_out_ref[...])
        node_ref[k] = out
        graph_ref[k] = jnp.sum(out, axis=0, keepdims=True) * (1.0 / n)


_G = 4


def kernel(x, adj, W_in, b_in, conv_W, conv_b, bn_gamma, bn_beta, bn_mean,
           bn_var, W_out, b_out):
    B, T, Cin = x.shape
    N = adj.shape[-1]
    H = W_in.shape[1]
    Cout = W_out.shape[1]
    L = conv_W.shape[0]
    BT = B * T

    g3 = lambda i: (i, 0, 0)
    rep2 = lambda i: (0, 0)
    rep3 = lambda i: (0, 0, 0)
    node, graph = pl.pallas_call(
        _fwd,
        grid=(BT // _G,),
        in_specs=[
            pl.BlockSpec((_G, 1, Cin), g3),
            pl.BlockSpec((_G, N, N), g3),
            pl.BlockSpec((Cin, H), rep2),
            pl.BlockSpec((1, H), rep2),
            pl.BlockSpec((L, H, H), rep3),
            pl.BlockSpec((L, H), rep2),
            pl.BlockSpec((L, H), rep2),
            pl.BlockSpec((L, H), rep2),
            pl.BlockSpec((L, H), rep2),
            pl.BlockSpec((L, H), rep2),
            pl.BlockSpec((H, Cout), rep2),
            pl.BlockSpec((1, Cout), rep2),
        ],
        out_specs=[
            pl.BlockSpec((_G, N, Cout), g3),
            pl.BlockSpec((_G, 1, Cout), g3),
        ],
        out_shape=[
            jax.ShapeDtypeStruct((BT, N, Cout), jnp.float32),
            jax.ShapeDtypeStruct((BT, 1, Cout), jnp.float32),
        ],
    )(x.reshape(BT, 1, Cin), adj.reshape(BT, N, N), W_in, b_in.reshape(1, H),
      conv_W, conv_b, bn_gamma, bn_beta, bn_mean, bn_var, W_out,
      b_out.reshape(1, Cout))
    return node.reshape(B, T, N, Cout), graph.reshape(B, T, Cout)
